# baseline (speedup 1.0000x reference)
import jax
import jax.numpy as jnp
from jax.experimental import pallas as pl

_BLK = 16384


def _rowsum64_sublane(sT):
    acc = sT[0:8, :]
    for a in range(1, 8):
        acc = acc + sT[8 * a:8 * a + 8, :]
    acc = acc[0:4, :] + acc[4:8, :]
    acc = acc[0:2, :] + acc[2:4, :]
    return acc[0:1, :] + acc[1:2, :]


def _rvq_body(x_ref, w_in_ref, b_in_ref, cb0_ref, cb1_ref,
              cb0T_ref, cb1T_ref, c20_ref, c21_ref,
              w_out_ref, b_out_ref, o_ref):
    x = x_ref[...]
    z = jnp.dot(x, w_in_ref[...], preferred_element_type=jnp.float32)
    z = z + b_in_ref[...]
    zT = z.T

    def nearest_code(rT, cb, cbT, c2):
        r2 = _rowsum64_sublane(rT * rT)
        distT = r2 - 2.0 * jnp.dot(cb, rT, preferred_element_type=jnp.float32)
        distT = distT + c2
        ind = jnp.argmin(distT, axis=0)
        onehotT = (jax.lax.broadcasted_iota(jnp.int32, distT.shape, 0)
                   == ind[None, :]).astype(jnp.bfloat16)
        hi = cbT.astype(jnp.bfloat16)
        rem = cbT - hi.astype(jnp.float32)
        mid = rem.astype(jnp.bfloat16)
        lo = (rem - mid.astype(jnp.float32)).astype(jnp.bfloat16)
        sel = lambda limb: jnp.dot(limb, onehotT,
                                   preferred_element_type=jnp.float32)
        return (sel(hi) + sel(mid)) + sel(lo)

    code0T = nearest_code(zT, cb0_ref[...], cb0T_ref[...], c20_ref[...])
    code1T = nearest_code(zT - code0T, cb1_ref[...], cb1T_ref[...],
                          c21_ref[...])
    zq = (code0T + code1T).T
    out = jnp.dot(zq, w_out_ref[...], preferred_element_type=jnp.float32)
    o_ref[...] = out + b_out_ref[...]


def _codebook_sqnorm(cb):
    s = cb * cb
    w = s.reshape(cb.shape[0], 8, 8)
    acc = w[:, 0, :]
    for a in range(1, 8):
        acc = acc + w[:, a, :]
    acc = acc[:, 0:4] + acc[:, 4:8]
    acc = acc[:, 0:2] + acc[:, 2:4]
    return acc[:, 0:1] + acc[:, 1:2]


def kernel(mel_frame, W_in, b_in, cb0, cb1, W_out, b_out):
    b, t, d_in = mel_frame.shape
    n = b * t
    d = W_in.shape[1]
    k = cb0.shape[0]
    x = mel_frame.reshape(n, d_in)
    c20 = _codebook_sqnorm(cb0)
    c21 = _codebook_sqnorm(cb1)
    full = lambda shape: pl.BlockSpec(shape, lambda i: (0, 0))
    out = pl.pallas_call(
        _rvq_body,
        grid=(n // _BLK,),
        in_specs=[
            pl.BlockSpec((_BLK, d_in), lambda i: (i, 0)),
            full((d_in, d)),
            full((1, d)),
            full((k, d)),
            full((k, d)),
            full((d, k)),
            full((d, k)),
            full((k, 1)),
            full((k, 1)),
            full((d, d_in)),
            full((1, d_in)),
        ],
        out_specs=pl.BlockSpec((_BLK, d_in), lambda i: (i, 0)),
        out_shape=jax.ShapeDtypeStruct((n, d_in), jnp.float32),
    )(x, W_in, b_in.reshape(1, d), cb0, cb1, cb0.T, cb1.T,
      c20, c21, W_out, b_out.reshape(1, d_in))
    return out.reshape(b, t, d_in)


# speedup vs baseline: 1.0740x; 1.0740x over previous
import jax
import jax.numpy as jnp
from jax.experimental import pallas as pl

_BLK = 8192


def _rowsum64_sublane(sT):
    acc = sT[0:8, :]
    for a in range(1, 8):
        acc = acc + sT[8 * a:8 * a + 8, :]
    acc = acc[0:4, :] + acc[4:8, :]
    acc = acc[0:2, :] + acc[2:4, :]
    return acc[0:1, :] + acc[1:2, :]


def _rvq_body(x_ref, w_in_ref, b_in_ref, cb0_ref, cb1_ref,
              hi0_ref, mid0_ref, lo0_ref, hi1_ref, mid1_ref, lo1_ref,
              c20_ref, c21_ref, w_out_ref, b_out_ref, o_ref):
    x = x_ref[...]
    z = jnp.dot(x, w_in_ref[...], preferred_element_type=jnp.float32)
    z = z + b_in_ref[...]
    zT = z.T

    def nearest_code(rT, cb, hi, mid, lo, c2):
        r2 = _rowsum64_sublane(rT * rT)
        distT = r2 - 2.0 * jnp.dot(cb, rT, preferred_element_type=jnp.float32)
        distT = distT + c2
        ind = jnp.argmin(distT, axis=0)
        onehotT = (jax.lax.broadcasted_iota(jnp.int32, distT.shape, 0)
                   == ind[None, :]).astype(jnp.bfloat16)
        sel = lambda limb: jnp.dot(limb, onehotT,
                                   preferred_element_type=jnp.float32)
        return (sel(hi) + sel(mid)) + sel(lo)

    code0T = nearest_code(zT, cb0_ref[...], hi0_ref[...], mid0_ref[...],
                          lo0_ref[...], c20_ref[...])
    code1T = nearest_code(zT - code0T, cb1_ref[...], hi1_ref[...],
                          mid1_ref[...], lo1_ref[...], c21_ref[...])
    zq = (code0T + code1T).T
    out = jnp.dot(zq, w_out_ref[...], preferred_element_type=jnp.float32)
    o_ref[...] = out + b_out_ref[...]


def _limbsT(cb):
    mask = jnp.uint32(0xFFFF0000)
    bits = jax.lax.bitcast_convert_type(cb, jnp.uint32)
    hi = jax.lax.bitcast_convert_type(bits & mask, jnp.float32)
    rem = cb - hi
    rbits = jax.lax.bitcast_convert_type(rem, jnp.uint32)
    mid = jax.lax.bitcast_convert_type(rbits & mask, jnp.float32)
    lo = rem - mid
    cast = lambda v: v.T.astype(jnp.bfloat16)
    return cast(hi), cast(mid), cast(lo)


def _codebook_sqnorm(cb):
    s = cb * cb
    w = s.reshape(cb.shape[0], 8, 8)
    acc = w[:, 0, :]
    for a in range(1, 8):
        acc = acc + w[:, a, :]
    acc = acc[:, 0:4] + acc[:, 4:8]
    acc = acc[:, 0:2] + acc[:, 2:4]
    return acc[:, 0:1] + acc[:, 1:2]


def kernel(mel_frame, W_in, b_in, cb0, cb1, W_out, b_out):
    b, t, d_in = mel_frame.shape
    n = b * t
    d = W_in.shape[1]
    k = cb0.shape[0]
    x = mel_frame.reshape(n, d_in)
    hi0, mid0, lo0 = _limbsT(cb0)
    hi1, mid1, lo1 = _limbsT(cb1)
    c20 = _codebook_sqnorm(cb0)
    c21 = _codebook_sqnorm(cb1)
    full = lambda shape: pl.BlockSpec(shape, lambda i: (0, 0))
    out = pl.pallas_call(
        _rvq_body,
        grid=(n // _BLK,),
        in_specs=[
            pl.BlockSpec((_BLK, d_in), lambda i: (i, 0)),
            full((d_in, d)),
            full((1, d)),
            full((k, d)),
            full((k, d)),
            full((d, k)), full((d, k)), full((d, k)),
            full((d, k)), full((d, k)), full((d, k)),
            full((k, 1)),
            full((k, 1)),
            full((d, d_in)),
            full((1, d_in)),
        ],
        out_specs=pl.BlockSpec((_BLK, d_in), lambda i: (i, 0)),
        out_shape=jax.ShapeDtypeStruct((n, d_in), jnp.float32),
    )(x, W_in, b_in.reshape(1, d), cb0, cb1,
      hi0, mid0, lo0, hi1, mid1, lo1,
      c20, c21, W_out, b_out.reshape(1, d_in))
    return out.reshape(b, t, d_in)
